# baseline (speedup 1.0000x reference)
import functools

import jax
import jax.numpy as jnp
from jax.experimental import pallas as pl
from jax.experimental.pallas import tpu as pltpu

N = 1024
VEC = 256
OUT = 64
EDGE = 4
JL = 128 // EDGE
FR = N * N // JL
BLK = 1024
IB = BLK // JL
GRID = FR // BLK


def _body(x_ref, adj_ref, adjt_ref, fin_ref, fout_ref, wbi_ref, wbo_ref,
          wni_ref, wno_ref, wei_ref, weo_ref, wagg_ref, bias_ref,
          sela_ref, selb_ref, s4_ref, selc_ref, maskc_ref,
          out_ref, min_ref, mout_ref,
          sin_s, sout_s, nin_s, nout_s, eins_s, eouta_s):
    i = pl.program_id(0)
    rows = pl.ds(i * IB, IB)

    @pl.when(i == 0)
    def _init():
        xv = x_ref[...]
        sin_s[...] = jnp.dot(xv, wni_ref[...], preferred_element_type=jnp.float32)
        sout_s[...] = jnp.dot(xv, wno_ref[...], preferred_element_type=jnp.float32)
        eouta_s[...] = jnp.zeros_like(eouta_s)

    fin = fin_ref[...]
    min_ref[...] = jnp.dot(fin, wbi_ref[...],
                           preferred_element_type=jnp.float32)
    fin_i = jnp.dot(sela_ref[...], fin, preferred_element_type=jnp.float32)
    fin_ie = jnp.dot(fin_i, s4_ref[...], preferred_element_type=jnp.float32)
    eins_s[rows, :] = jnp.dot(fin_ie, wei_ref[...],
                              preferred_element_type=jnp.float32)

    fout = fout_ref[...]
    mout_ref[...] = jnp.dot(fout, wbo_ref[...],
                            preferred_element_type=jnp.float32)
    eouta_s[...] += jnp.dot(selb_ref[...], fout,
                            preferred_element_type=jnp.float32)

    nout_s[rows, :] = jnp.dot(adj_ref[...], sout_s[...],
                              preferred_element_type=jnp.float32)
    nin_s[rows, :] = jnp.dot(adjt_ref[...], sin_s[...],
                             preferred_element_type=jnp.float32)

    @pl.when(i == GRID - 1)
    def _finish():
        tmp = jnp.dot(selc_ref[...], eouta_s[...],
                      preferred_element_type=jnp.float32)
        colsum = jnp.dot(tmp * maskc_ref[...], s4_ref[...],
                         preferred_element_type=jnp.float32)
        eout = jnp.dot(colsum, weo_ref[...], preferred_element_type=jnp.float32)
        wagg = wagg_ref[...]
        h = OUT // 2
        acc = jnp.dot(nin_s[...], wagg[0:h, :],
                      preferred_element_type=jnp.float32)
        acc += jnp.dot(nout_s[...], wagg[h:2 * h, :],
                       preferred_element_type=jnp.float32)
        acc += jnp.dot(eins_s[...], wagg[2 * h:2 * h + OUT, :],
                       preferred_element_type=jnp.float32)
        acc += jnp.dot(eout, wagg[2 * h + OUT:, :],
                       preferred_element_type=jnp.float32)
        out_ref[...] = jnp.maximum(acc + bias_ref[...], 0.0)


@jax.jit
def kernel(x, adj_matrix, edge_in_feat_matrix, edge_out_feat_matrix,
           weight_node_in, weight_node_out, weight_edge_in, weight_edge_out,
           weight_aggressive, bias):
    adj_t = adj_matrix.T
    bias2 = bias.reshape(1, OUT)
    fin2 = edge_in_feat_matrix.reshape(FR, 128)
    fout2 = edge_out_feat_matrix.reshape(FR, 128)

    rows_i = jnp.arange(128)
    cols_i = jnp.arange(JL * OUT)
    blk_match = (rows_i[:, None] // EDGE) == (cols_i[None, :] // OUT)

    def make_big(w):
        full = w[rows_i % EDGE][:, cols_i % OUT]
        return jnp.where(blk_match, full, 0.0)

    wbig_in = make_big(weight_edge_in)
    wbig_out = make_big(weight_edge_out)

    sela = jnp.kron(jnp.eye(IB, dtype=jnp.float32),
                    jnp.ones((1, JL), jnp.float32))
    selb = jnp.kron(jnp.ones((1, IB), jnp.float32),
                    jnp.eye(JL, dtype=jnp.float32))
    s4 = jnp.kron(jnp.ones((JL, 1), jnp.float32),
                  jnp.eye(EDGE, dtype=jnp.float32))
    selc = jnp.kron(jnp.eye(JL, dtype=jnp.float32),
                    jnp.ones((JL, 1), jnp.float32))
    maskc = ((jnp.arange(N)[:, None] % JL) ==
             (jnp.arange(128)[None, :] // EDGE)).astype(jnp.float32)

    in_specs = [
        pl.BlockSpec((N, VEC), lambda i: (0, 0)),
        pl.BlockSpec((IB, N), lambda i: (i, 0)),
        pl.BlockSpec((IB, N), lambda i: (i, 0)),
        pl.BlockSpec((BLK, 128), lambda i: (i, 0)),
        pl.BlockSpec((BLK, 128), lambda i: (i, 0)),
        pl.BlockSpec((128, JL * OUT), lambda i: (0, 0)),
        pl.BlockSpec((128, JL * OUT), lambda i: (0, 0)),
        pl.BlockSpec((VEC, OUT // 2), lambda i: (0, 0)),
        pl.BlockSpec((VEC, OUT // 2), lambda i: (0, 0)),
        pl.BlockSpec((EDGE, OUT), lambda i: (0, 0)),
        pl.BlockSpec((EDGE, OUT), lambda i: (0, 0)),
        pl.BlockSpec((3 * OUT, OUT), lambda i: (0, 0)),
        pl.BlockSpec((1, OUT), lambda i: (0, 0)),
        pl.BlockSpec((IB, BLK), lambda i: (0, 0)),
        pl.BlockSpec((JL, BLK), lambda i: (0, 0)),
        pl.BlockSpec((128, EDGE), lambda i: (0, 0)),
        pl.BlockSpec((N, JL), lambda i: (0, 0)),
        pl.BlockSpec((N, 128), lambda i: (0, 0)),
    ]
    out_specs = [
        pl.BlockSpec((N, OUT), lambda i: (0, 0)),
        pl.BlockSpec((BLK, JL * OUT), lambda i: (i, 0)),
        pl.BlockSpec((BLK, JL * OUT), lambda i: (i, 0)),
    ]

    out, min2, mout2 = pl.pallas_call(
        _body,
        grid=(GRID,),
        in_specs=in_specs,
        out_specs=out_specs,
        out_shape=[
            jax.ShapeDtypeStruct((N, OUT), jnp.float32),
            jax.ShapeDtypeStruct((FR, JL * OUT), jnp.float32),
            jax.ShapeDtypeStruct((FR, JL * OUT), jnp.float32),
        ],
        scratch_shapes=[
            pltpu.VMEM((N, OUT // 2), jnp.float32),
            pltpu.VMEM((N, OUT // 2), jnp.float32),
            pltpu.VMEM((N, OUT // 2), jnp.float32),
            pltpu.VMEM((N, OUT // 2), jnp.float32),
            pltpu.VMEM((N, OUT), jnp.float32),
            pltpu.VMEM((JL, 128), jnp.float32),
        ],
        compiler_params=pltpu.CompilerParams(
            dimension_semantics=("arbitrary",),
        ),
    )(x, adj_matrix, adj_t, fin2, fout2, wbig_in, wbig_out,
      weight_node_in, weight_node_out, weight_edge_in, weight_edge_out,
      weight_aggressive, bias2, sela, selb, s4, selc, maskc)

    return (out,
            min2.reshape(N, N, OUT),
            mout2.reshape(N, N, OUT))


# speedup vs baseline: 2.0807x; 2.0807x over previous
import functools

import jax
import jax.numpy as jnp
from jax.experimental import pallas as pl
from jax.experimental.pallas import tpu as pltpu

N = 1024
VEC = 256
OUT = 64
EDGE = 4
IB = 8
GRID = N // IB


def _body(x_ref, adj_ref, adjt_ref, fin_ref, fout_ref,
          wni_ref, wno_ref, wei_ref, weo_ref, wagg_ref, bias_ref,
          out_ref, min_ref, mout_ref,
          sin_s, sout_s, nin_s, nout_s, eins_s, eouta_s):
    i = pl.program_id(0)
    rows = pl.ds(i * IB, IB)

    @pl.when(i == 0)
    def _init():
        xv = x_ref[...]
        sin_s[...] = jnp.dot(xv, wni_ref[...], preferred_element_type=jnp.float32)
        sout_s[...] = jnp.dot(xv, wno_ref[...], preferred_element_type=jnp.float32)
        eouta_s[...] = jnp.zeros_like(eouta_s)

    wei = wei_ref[...]
    weo = weo_ref[...]

    fin = fin_ref[...]
    min_ref[...] = jnp.dot(
        fin.reshape(IB * N, EDGE), wei,
        preferred_element_type=jnp.float32).reshape(IB, N, OUT)
    eins_s[rows, :] = jnp.dot(fin.sum(axis=1), wei,
                              preferred_element_type=jnp.float32)

    fout = fout_ref[...]
    mout_ref[...] = jnp.dot(
        fout.reshape(IB * N, EDGE), weo,
        preferred_element_type=jnp.float32).reshape(IB, N, OUT)
    eouta_s[...] += fout.sum(axis=0)

    nout_s[rows, :] = jnp.dot(adj_ref[...], sout_s[...],
                              preferred_element_type=jnp.float32)
    nin_s[rows, :] = jnp.dot(adjt_ref[...], sin_s[...],
                             preferred_element_type=jnp.float32)

    @pl.when(i == GRID - 1)
    def _finish():
        eout = jnp.dot(eouta_s[...], weo, preferred_element_type=jnp.float32)
        wagg = wagg_ref[...]
        h = OUT // 2
        acc = jnp.dot(nin_s[...], wagg[0:h, :],
                      preferred_element_type=jnp.float32)
        acc += jnp.dot(nout_s[...], wagg[h:2 * h, :],
                       preferred_element_type=jnp.float32)
        acc += jnp.dot(eins_s[...], wagg[2 * h:2 * h + OUT, :],
                       preferred_element_type=jnp.float32)
        acc += jnp.dot(eout, wagg[2 * h + OUT:, :],
                       preferred_element_type=jnp.float32)
        out_ref[...] = jnp.maximum(acc + bias_ref[...], 0.0)


@jax.jit
def kernel(x, adj_matrix, edge_in_feat_matrix, edge_out_feat_matrix,
           weight_node_in, weight_node_out, weight_edge_in, weight_edge_out,
           weight_aggressive, bias):
    adj_t = adj_matrix.T
    bias2 = bias.reshape(1, OUT)

    in_specs = [
        pl.BlockSpec((N, VEC), lambda i: (0, 0)),
        pl.BlockSpec((IB, N), lambda i: (i, 0)),
        pl.BlockSpec((IB, N), lambda i: (i, 0)),
        pl.BlockSpec((IB, N, EDGE), lambda i: (i, 0, 0)),
        pl.BlockSpec((IB, N, EDGE), lambda i: (i, 0, 0)),
        pl.BlockSpec((VEC, OUT // 2), lambda i: (0, 0)),
        pl.BlockSpec((VEC, OUT // 2), lambda i: (0, 0)),
        pl.BlockSpec((EDGE, OUT), lambda i: (0, 0)),
        pl.BlockSpec((EDGE, OUT), lambda i: (0, 0)),
        pl.BlockSpec((3 * OUT, OUT), lambda i: (0, 0)),
        pl.BlockSpec((1, OUT), lambda i: (0, 0)),
    ]
    out_specs = [
        pl.BlockSpec((N, OUT), lambda i: (0, 0)),
        pl.BlockSpec((IB, N, OUT), lambda i: (i, 0, 0)),
        pl.BlockSpec((IB, N, OUT), lambda i: (i, 0, 0)),
    ]

    out, min3, mout3 = pl.pallas_call(
        _body,
        grid=(GRID,),
        in_specs=in_specs,
        out_specs=out_specs,
        out_shape=[
            jax.ShapeDtypeStruct((N, OUT), jnp.float32),
            jax.ShapeDtypeStruct((N, N, OUT), jnp.float32),
            jax.ShapeDtypeStruct((N, N, OUT), jnp.float32),
        ],
        scratch_shapes=[
            pltpu.VMEM((N, OUT // 2), jnp.float32),
            pltpu.VMEM((N, OUT // 2), jnp.float32),
            pltpu.VMEM((N, OUT // 2), jnp.float32),
            pltpu.VMEM((N, OUT // 2), jnp.float32),
            pltpu.VMEM((N, OUT), jnp.float32),
            pltpu.VMEM((N, EDGE), jnp.float32),
        ],
        compiler_params=pltpu.CompilerParams(
            dimension_semantics=("arbitrary",),
        ),
    )(x, adj_matrix, adj_t, edge_in_feat_matrix, edge_out_feat_matrix,
      weight_node_in, weight_node_out, weight_edge_in, weight_edge_out,
      weight_aggressive, bias2)

    return out, min3, mout3
